# baseline (speedup 1.0000x reference)
import functools

import jax
import jax.numpy as jnp
from jax import lax
from jax.experimental import pallas as pl
from jax.experimental.pallas import tpu as pltpu
from jax.experimental.pallas import tpu_sc as plsc

B, T, F = 1024, 200, 128
EMB = 64
OUTW = (F - 1) + EMB
NC, NS, L = 2, 16, 16
NW = NC * NS
TS_T, TS_B = 8, 128
NST = (T // TS_T) * (B // TS_B)
SPW = (NST + NW - 1) // NW


def _ids_body(x_ref, ids_ref):
    v = x_ref[:, :, 0]
    ids_ref[...] = (v - 1.0).astype(jnp.int32).T


def _ids_kernel(x):
    bb = 128
    return pl.pallas_call(
        _ids_body,
        grid=(B // bb,),
        in_specs=[pl.BlockSpec((bb, T, F), lambda i: (i, 0, 0))],
        out_specs=pl.BlockSpec((T, bb), lambda i: (0, i)),
        out_shape=jax.ShapeDtypeStruct((T, B), jnp.int32),
    )(x)


def _sc_body(table_hbm, ids_hbm, out_hbm,
             idxv, gb0, gb1, gb2, stg, g0, g1, g2, ssem):
    cid = lax.axis_index("c")
    sid = lax.axis_index("s")
    wid = sid * NC + cid

    gbufs = (gb0, gb1, gb2)
    gsems = (g0, g1, g2)
    iota16 = lax.iota(jnp.int32, L)
    zeros16 = jnp.zeros((L,), jnp.int32)

    def gath(tt):
        gb = tt % 3
        return pltpu.make_async_copy(
            table_hbm.at[idxv.at[tt]], gbufs[gb], gsems[gb])

    def transpose_tt(tt):
        gbuf = gbufs[tt % 3]

        @plsc.parallel_loop(0, EMB, step=1, unroll=4)
        def _feat(e):
            esplat = zeros16 + e
            vs = [plsc.load_gather(gbuf, [iota16 + k * L, esplat])
                  for k in range(TS_B // L)]
            for k in range(TS_B // L):
                stg[e, tt, pl.ds(k * L, L)] = vs[k]

    def stg_copy(t0, b0):
        return pltpu.make_async_copy(
            stg,
            out_hbm.at[pl.ds(F - 1, EMB), pl.ds(t0, TS_T), pl.ds(b0, TS_B)],
            ssem)

    def supertile(s, carry):
        st = wid + NW * s

        @pl.when(st < NST)
        def _():
            t0 = (st // (B // TS_B)) * TS_T
            b0 = (st % (B // TS_B)) * TS_B
            pltpu.sync_copy(
                ids_hbm.at[pl.ds(t0, TS_T), pl.ds(b0, TS_B)], idxv)
            gath(0).start()
            gath(1).start()
            gath(2).start()

            @pl.when(s > 0)
            def _():
                stp = st - NW
                pt0 = (stp // (B // TS_B)) * TS_T
                pb0 = (stp % (B // TS_B)) * TS_B
                stg_copy(pt0, pb0).wait()

            for tt in range(TS_T):
                gath(tt).wait()
                transpose_tt(tt)
                if tt + 3 < TS_T:
                    gath(tt + 3).start()
            stg_copy(t0, b0).start()
        return carry

    lax.fori_loop(0, SPW, supertile, 0)
    s_last = (NST - 1 - wid) // NW
    st_last = wid + NW * s_last
    lt0 = (st_last // (B // TS_B)) * TS_T
    lb0 = (st_last % (B // TS_B)) * TS_B
    stg_copy(lt0, lb0).wait()


def _sc_kernel(table128, ids_t):
    mesh = plsc.VectorSubcoreMesh(core_axis_name="c", subcore_axis_name="s")
    return pl.kernel(
        _sc_body,
        mesh=mesh,
        compiler_params=pltpu.CompilerParams(needs_layout_passes=False),
        out_type=jax.ShapeDtypeStruct((OUTW, T, B), jnp.float32),
        scratch_types=[
            pltpu.VMEM((TS_T, TS_B), jnp.int32),
            pltpu.VMEM((TS_B, F), jnp.float32),
            pltpu.VMEM((TS_B, F), jnp.float32),
            pltpu.VMEM((TS_B, F), jnp.float32),
            pltpu.VMEM((EMB, TS_T, TS_B), jnp.float32),
            pltpu.SemaphoreType.DMA,
            pltpu.SemaphoreType.DMA,
            pltpu.SemaphoreType.DMA,
            pltpu.SemaphoreType.DMA,
        ],
    )(table128, ids_t)


def _xpose_body(x_ref, outt_in_ref, out_ref):
    del outt_in_ref
    for tt in range(x_ref.shape[1]):
        xt = x_ref[:, tt, :].T
        out_ref[:, tt, :] = xt[1:, :]


def _xpose_kernel(x, out_t):
    tb, bb = 40, 256
    return pl.pallas_call(
        _xpose_body,
        grid=(T // tb, B // bb),
        in_specs=[
            pl.BlockSpec((bb, tb, F), lambda t, b: (b, t, 0)),
            pl.BlockSpec(memory_space=pltpu.MemorySpace.HBM),
        ],
        out_specs=pl.BlockSpec((F - 1, tb, bb), lambda t, b: (0, t, b)),
        out_shape=jax.ShapeDtypeStruct((OUTW, T, B), jnp.float32),
        input_output_aliases={1: 0},
    )(x, out_t)


@jax.jit
def _run(x, table):
    table128 = jnp.pad(table, ((0, 0), (0, F - EMB)))
    ids_t = _ids_kernel(x)
    out_t = _sc_kernel(table128, ids_t)
    out_t = _xpose_kernel(x, out_t)
    return jnp.transpose(out_t, (2, 1, 0))


def kernel(x, table):
    return _run(x, table)


# speedup vs baseline: 1.1809x; 1.1809x over previous
import functools

import jax
import jax.numpy as jnp
from jax import lax
from jax.experimental import pallas as pl
from jax.experimental.pallas import tpu as pltpu
from jax.experimental.pallas import tpu_sc as plsc

B, T, F = 1024, 200, 128
EMB = 64
N = B * T
OUTW = (F - 1) + EMB
NC, NS, L = 2, 16, 16
NW = NC * NS
RPW = N // NW
C = 64
NCHUNK = RPW // C


def _sc_body(x_hbm, table_hbm, out_hbm,
             xc0, xc1, idx0, idx1, emb0, emb1, oc0, oc1,
             ld0, ld1, g0, g1, s0, s1):
    cid = lax.axis_index("c")
    sid = lax.axis_index("s")
    wid = sid * NC + cid
    wbase = wid * RPW

    xcs, idxs, embs, ocs = (xc0, xc1), (idx0, idx1), (emb0, emb1), (oc0, oc1)
    lds, gs, ss = (ld0, ld1), (g0, g1), (s0, s1)

    zeros16 = jnp.zeros((L,), jnp.int32)
    iota16 = lax.iota(jnp.int32, L)
    shl1 = jnp.where(iota16 < 15, iota16 + 1, 15)

    def load(j, b):
        return pltpu.make_async_copy(
            x_hbm.at[pl.ds(wbase + j * C, C)], xcs[b], lds[b])

    def gath(b):
        return pltpu.make_async_copy(table_hbm.at[idxs[b]], embs[b], gs[b])

    def store(j, b):
        return pltpu.make_async_copy(
            ocs[b], out_hbm.at[pl.ds(wbase + j * C, C)], ss[b])

    def idx_compute(b):
        for jj in range(C // L):
            rows = iota16 + jj * L
            v = plsc.load_gather(xcs[b], [rows, zeros16])
            idxs[b][pl.ds(jj * L, L)] = (v - 1.0).astype(jnp.int32)

    def assemble(b):
        xc, emb, outc = xcs[b], embs[b], ocs[b]

        @plsc.parallel_loop(0, C, step=1, unroll=4)
        def _row(r):
            vxs = [xc[r, pl.ds(1 + j * L, L)] for j in range(7)]
            vx = xc[r, pl.ds(112, L)]
            ve = emb[r, pl.ds(0, L)]
            ves = [emb[r, pl.ds(o, L)] for o in (1, 17, 33, 48)]
            shifted = vx.at[shl1].get(mode="promise_in_bounds")
            splat0 = ve.at[zeros16].get(mode="promise_in_bounds")
            vmix = jnp.where(iota16 < 15, shifted, splat0)
            for j in range(7):
                outc[r, pl.ds(j * L, L)] = vxs[j]
            outc[r, pl.ds(112, L)] = vmix
            for k, o in enumerate((128, 144, 160, 175)):
                outc[r, pl.ds(o, L)] = ves[k]

    load(0, 0).start()
    load(0, 0).wait()
    idx_compute(0)
    gath(0).start()
    load(1, 1).start()

    def pair(k, carry):
        for b in (0, 1):
            j = 2 * k + b
            nb = 1 - b

            @pl.when(j + 1 < NCHUNK)
            def _():
                load(j + 1, nb).wait()
                idx_compute(nb)
                gath(nb).start()

            gath(b).wait()

            @pl.when(j >= 2)
            def _():
                store(j - 2, b).wait()

            assemble(b)
            store(j, b).start()

            @pl.when(j + 2 < NCHUNK)
            def _():
                load(j + 2, b).start()
        return carry

    lax.fori_loop(0, NCHUNK // 2, pair, 0)
    store(NCHUNK - 2, 0).wait()
    store(NCHUNK - 1, 1).wait()


@jax.jit
def _run(x2d, table):
    mesh = plsc.VectorSubcoreMesh(core_axis_name="c", subcore_axis_name="s")
    return pl.kernel(
        _sc_body,
        mesh=mesh,
        compiler_params=pltpu.CompilerParams(needs_layout_passes=False),
        out_type=jax.ShapeDtypeStruct((N, OUTW), jnp.float32),
        scratch_types=[
            pltpu.VMEM((C, F), jnp.float32),
            pltpu.VMEM((C, F), jnp.float32),
            pltpu.VMEM((C,), jnp.int32),
            pltpu.VMEM((C,), jnp.int32),
            pltpu.VMEM((C, F), jnp.float32),
            pltpu.VMEM((C, F), jnp.float32),
            pltpu.VMEM((C, OUTW), jnp.float32),
            pltpu.VMEM((C, OUTW), jnp.float32),
            pltpu.SemaphoreType.DMA,
            pltpu.SemaphoreType.DMA,
            pltpu.SemaphoreType.DMA,
            pltpu.SemaphoreType.DMA,
            pltpu.SemaphoreType.DMA,
            pltpu.SemaphoreType.DMA,
        ],
    )(x2d, table)


def kernel(x, table):
    table128 = jnp.pad(table, ((0, 0), (0, F - EMB)))
    out = _run(x.reshape(N, F), table128)
    return out.reshape(B, T, OUTW)
